# baseline (speedup 1.0000x reference)
import functools

import jax
import jax.numpy as jnp
from jax import lax
from jax.experimental import pallas as pl
from jax.experimental.pallas import tpu as pltpu
from jax.experimental.pallas import tpu_sc as plsc

_LANES = 16
_NC = 2
_NS = 16
_DT = 1


@functools.partial(jax.jit, static_argnums=(2, 3, 4, 5))
def _select_t(x, indices, B, T, C, K):
    n_workers = _NC * _NS
    bw = B // n_workers
    bwp = bw + 16
    dt = _DT
    n_steps = T // dt
    n_groups = K // _LANES

    mesh = plsc.VectorSubcoreMesh(
        core_axis_name="c", subcore_axis_name="s",
        num_cores=_NC, num_subcores=_NS)

    @functools.partial(
        pl.kernel,
        out_type=jax.ShapeDtypeStruct((T, K, B), jnp.float32),
        mesh=mesh,
        scratch_types=[
            pltpu.VMEM((K,), jnp.int32),
            pltpu.VMEM((2, dt, bw), jnp.int32),
            pltpu.VMEM((2, dt, bw, C), jnp.float32),
            pltpu.VMEM((2, dt, K, bwp), jnp.float32),
            pltpu.SemaphoreType.DMA,
            pltpu.SemaphoreType.DMA,
            pltpu.SemaphoreType.DMA,
            pltpu.SemaphoreType.DMA,
        ],
        compiler_params=pltpu.CompilerParams(needs_layout_passes=False),
    )
    def body(x_hbm, idx_hbm, out_hbm, idx_v, rid_v, in_v, out_v,
             sin0, sin1, sout0, sout1):
        sin = (sin0, sin1)
        sout = (sout0, sout1)
        wid = lax.axis_index("s") * _NC + lax.axis_index("c")
        b0 = wid * bw
        pltpu.sync_copy(idx_hbm, idx_v)

        lane = lax.iota(jnp.int32, _LANES)
        mvecs = [(lane + g * _LANES) * T for g in range(n_groups * bw // K)]
        idx_vecs = [idx_v[pl.ds(g * _LANES, _LANES)] for g in range(n_groups)]
        jvecs = [lane + g * _LANES for g in range(n_groups)]

        def start_in(i, s):
            t0 = i * dt
            for u in range(dt):
                for g in range(bw // _LANES):
                    rid_v[s, u, pl.ds(g * _LANES, _LANES)] = (
                        mvecs[g] + (b0 * T + t0 + u))
            for u in range(dt):
                pltpu.async_copy(x_hbm.at[rid_v.at[s, u]], in_v.at[s, u],
                                 sin[s])

        def wait_in(i, s):
            for u in range(dt):
                pltpu.make_async_copy(x_hbm.at[rid_v.at[s, u]],
                                      in_v.at[s, u], sin[s]).wait()

        def out_slice(i):
            return out_hbm.at[pl.ds(i * dt, dt), :, pl.ds(b0, bw)]

        def start_out(i, s):
            pltpu.async_copy(out_v.at[s, :, :, : bw], out_slice(i), sout[s])

        def wait_out(i, s):
            pltpu.make_async_copy(out_v.at[s, :, :, : bw], out_slice(i),
                                  sout[s]).wait()

        def compute(s):
            for u in range(dt):
                @plsc.parallel_loop(0, bw, unroll=4)
                def _(m):
                    mv = jnp.full((_LANES,), m, jnp.int32)
                    for g in range(n_groups):
                        vals = plsc.load_gather(in_v.at[s, u],
                                                [mv, idx_vecs[g]])
                        plsc.store_scatter(out_v.at[s, u], [jvecs[g], mv],
                                           vals)

        start_in(0, 0)
        start_in(1, 1)
        for s in (0, 1):
            wait_in(s, s)
            compute(s)
            start_out(s, s)
            start_in(s + 2, s)

        def loop_body(i2, carry):
            for s in (0, 1):
                i = 2 * i2 + s
                wait_in(i, s)
                wait_out(i - 2, s)
                compute(s)
                start_out(i, s)

                @pl.when(i2 < n_steps // 2 - 1)
                def _():
                    start_in(i + 2, s)

            return carry

        lax.fori_loop(1, n_steps // 2, loop_body, 0)

        wait_out(n_steps - 2, 0)
        wait_out(n_steps - 1, 1)

    return body(x, indices)


def kernel(x, indices):
    B, T, C = x.shape
    K = indices.shape[0]
    out_t = _select_t(x.reshape(B * T, C), indices.astype(jnp.int32),
                      B, T, C, K)
    return jnp.transpose(out_t, (2, 0, 1))


# speedup vs baseline: 1.4921x; 1.4921x over previous
import functools

import jax
import jax.numpy as jnp
from jax import lax
from jax.experimental import pallas as pl
from jax.experimental.pallas import tpu as pltpu
from jax.experimental.pallas import tpu_sc as plsc

_LANES = 16
_NC = 2
_NS = 16
_CHUNK = 200


@functools.partial(jax.jit, static_argnums=(2, 3, 4))
def _select_rows(x, indices, R, C, K):
    n_workers = _NC * _NS
    rows_per_w = R // n_workers
    ch = _CHUNK
    n_chunks = rows_per_w // ch
    n2 = n_chunks // 2
    n_groups = K // _LANES

    mesh = plsc.VectorSubcoreMesh(
        core_axis_name="c", subcore_axis_name="s",
        num_cores=_NC, num_subcores=_NS)

    @functools.partial(
        pl.kernel,
        out_type=jax.ShapeDtypeStruct((R, K), jnp.float32),
        mesh=mesh,
        scratch_types=[
            pltpu.VMEM((K,), jnp.int32),
            pltpu.VMEM((2, ch, C), jnp.float32),
            pltpu.VMEM((2, ch, K), jnp.float32),
            pltpu.SemaphoreType.DMA,
            pltpu.SemaphoreType.DMA,
            pltpu.SemaphoreType.DMA,
            pltpu.SemaphoreType.DMA,
        ],
        compiler_params=pltpu.CompilerParams(needs_layout_passes=False),
    )
    def body(x_hbm, idx_hbm, out_hbm, idx_v, in_v, out_v,
             sin0, sin1, sout0, sout1):
        sin = (sin0, sin1)
        sout = (sout0, sout1)
        wid = lax.axis_index("s") * _NC + lax.axis_index("c")
        base = wid * rows_per_w
        pltpu.sync_copy(idx_hbm, idx_v)
        idx_vecs = [idx_v[pl.ds(g * _LANES, _LANES)] for g in range(n_groups)]

        def in_slice(i):
            return x_hbm.at[pl.ds(base + i * ch, ch)]

        def out_slice(i):
            return out_hbm.at[pl.ds(base + i * ch, ch)]

        def start_in(i, b):
            pltpu.async_copy(in_slice(i), in_v.at[b], sin[b])

        def wait_in(i, b):
            pltpu.make_async_copy(in_slice(i), in_v.at[b], sin[b]).wait()

        def start_out(i, b):
            pltpu.async_copy(out_v.at[b], out_slice(i), sout[b])

        def wait_out(i, b):
            pltpu.make_async_copy(out_v.at[b], out_slice(i), sout[b]).wait()

        def compute(b):
            @plsc.parallel_loop(0, ch, unroll=8)
            def _(r):
                rv = jnp.full((_LANES,), r, jnp.int32)
                for g in range(n_groups):
                    out_v[b, r, pl.ds(g * _LANES, _LANES)] = (
                        plsc.load_gather(in_v.at[b], [rv, idx_vecs[g]]))

        start_in(0, 0)
        start_in(1, 1)
        for b in (0, 1):
            wait_in(b, b)
            compute(b)
            start_out(b, b)
            start_in(b + 2, b)

        def loop_body(i2, carry):
            for b in (0, 1):
                i = 2 * i2 + b
                wait_in(i, b)
                wait_out(i - 2, b)
                compute(b)
                start_out(i, b)

                @pl.when(i2 < n2 - 1)
                def _():
                    start_in(i + 2, b)

            return carry

        lax.fori_loop(1, n2, loop_body, 0)

        wait_out(n_chunks - 2, 0)
        wait_out(n_chunks - 1, 1)

    return body(x, indices)


def kernel(x, indices):
    B, T, C = x.shape
    K = indices.shape[0]
    R = B * T
    out = _select_rows(x.reshape(R, C), indices.astype(jnp.int32), R, C, K)
    return out.reshape(B, T, K)
